# Initial kernel scaffold; baseline (speedup 1.0000x reference)
import jax
import jax.numpy as jnp
from jax.experimental import pallas as pl


def kernel(local_patten, long_range_patten, weighted_X, V):
    raise NotImplementedError("write your pallas kernel here")


# speedup vs baseline: 4.3975x; 4.3975x over previous
import jax
import jax.numpy as jnp
from jax.experimental import pallas as pl

H, N, DH = 8, 2048, 128
BR = 256
ALPHA_MIX = 0.5


def _att_kernel(x_full_ref, x_strip_ref, v_ref, loc_ref, lon_ref, out_ref):
    x_full = x_full_ref[0]
    x_strip = x_strip_ref[0]
    v = v_ref[0, :, :, 0]
    f1 = jnp.dot(x_strip, v[0], preferred_element_type=jnp.float32)
    f2 = jnp.dot(x_full, v[1], preferred_element_type=jnp.float32)
    s = jax.nn.sigmoid(f1[:, None] + f2[None, :])
    e = jnp.exp(s)
    el = e * loc_ref[0]
    eg = e * lon_ref[0]
    dl = jnp.sum(el, axis=1, keepdims=True)
    dg = jnp.sum(eg, axis=1, keepdims=True)
    out_ref[0] = (1.0 - ALPHA_MIX) * (el / dl) + ALPHA_MIX * (eg / dg)


def kernel(local_patten, long_range_patten, weighted_X, V):
    grid = (H, N // BR)
    return pl.pallas_call(
        _att_kernel,
        grid=grid,
        in_specs=[
            pl.BlockSpec((1, N, DH), lambda h, i: (h, 0, 0)),
            pl.BlockSpec((1, BR, DH), lambda h, i: (h, i, 0)),
            pl.BlockSpec((1, 2, DH, 1), lambda h, i: (h, 0, 0, 0)),
            pl.BlockSpec((1, BR, N), lambda h, i: (h, i, 0)),
            pl.BlockSpec((1, BR, N), lambda h, i: (h, i, 0)),
        ],
        out_specs=pl.BlockSpec((1, BR, N), lambda h, i: (h, i, 0)),
        out_shape=jax.ShapeDtypeStruct((H, N, N), jnp.float32),
    )(weighted_X, weighted_X, V, local_patten, long_range_patten)


# speedup vs baseline: 4.4221x; 1.0056x over previous
import jax
import jax.numpy as jnp
from jax.experimental import pallas as pl

H, N, DH = 8, 2048, 128
BR = 256
ALPHA_MIX = 0.5


def _att_kernel(x_full_ref, x_strip_ref, v_ref, loc_ref, lon_ref, out_ref):
    x_full = x_full_ref[0]
    x_strip = x_strip_ref[0]
    v = v_ref[0, :, :, 0]
    f1 = jnp.dot(x_strip, v[0], preferred_element_type=jnp.float32)
    f2 = jnp.dot(x_full, v[1], preferred_element_type=jnp.float32)
    s = jax.nn.sigmoid(f1[:, None] + f2[None, :])
    e = jnp.exp(s)
    el = e * loc_ref[0]
    eg = e * lon_ref[0]
    dl = jnp.sum(el, axis=1, keepdims=True)
    dg = jnp.sum(eg, axis=1, keepdims=True)
    rl = (1.0 - ALPHA_MIX) / dl
    rg = ALPHA_MIX / dg
    out_ref[0] = el * rl + eg * rg


def kernel(local_patten, long_range_patten, weighted_X, V):
    grid = (H, N // BR)
    return pl.pallas_call(
        _att_kernel,
        grid=grid,
        in_specs=[
            pl.BlockSpec((1, N, DH), lambda h, i: (h, 0, 0)),
            pl.BlockSpec((1, BR, DH), lambda h, i: (h, i, 0)),
            pl.BlockSpec((1, 2, DH, 1), lambda h, i: (h, 0, 0, 0)),
            pl.BlockSpec((1, BR, N), lambda h, i: (h, i, 0)),
            pl.BlockSpec((1, BR, N), lambda h, i: (h, i, 0)),
        ],
        out_specs=pl.BlockSpec((1, BR, N), lambda h, i: (h, i, 0)),
        out_shape=jax.ShapeDtypeStruct((H, N, N), jnp.float32),
    )(weighted_X, weighted_X, V, local_patten, long_range_patten)


# speedup vs baseline: 4.5158x; 1.0212x over previous
import jax
import jax.numpy as jnp
from jax.experimental import pallas as pl

H, N, DH = 8, 2048, 128
BR = 256
ALPHA_MIX = 0.5


def _scores_kernel(x_ref, v_ref, f_ref):
    x = x_ref[0]
    v = v_ref[0, :, :, 0]
    f_ref[0] = jax.lax.dot_general(
        v, x, (((1,), (1,)), ((), ())),
        preferred_element_type=jnp.float32)


def _att_kernel(f_ref, loc_ref, lon_ref, out_ref):
    i = pl.program_id(1)
    f1 = f_ref[0, 0, pl.ds(i * BR, BR)]
    f2 = f_ref[0, 1, :]
    s = jax.nn.sigmoid(f1[:, None] + f2[None, :])
    e = jnp.exp(s)
    el = e * loc_ref[0]
    eg = e * lon_ref[0]
    dl = jnp.sum(el, axis=1, keepdims=True)
    dg = jnp.sum(eg, axis=1, keepdims=True)
    rl = (1.0 - ALPHA_MIX) / dl
    rg = ALPHA_MIX / dg
    out_ref[0] = el * rl + eg * rg


def kernel(local_patten, long_range_patten, weighted_X, V):
    f = pl.pallas_call(
        _scores_kernel,
        grid=(H,),
        in_specs=[
            pl.BlockSpec((1, N, DH), lambda h: (h, 0, 0)),
            pl.BlockSpec((1, 2, DH, 1), lambda h: (h, 0, 0, 0)),
        ],
        out_specs=pl.BlockSpec((1, 2, N), lambda h: (h, 0, 0)),
        out_shape=jax.ShapeDtypeStruct((H, 2, N), jnp.float32),
    )(weighted_X, V)

    return pl.pallas_call(
        _att_kernel,
        grid=(H, N // BR),
        in_specs=[
            pl.BlockSpec((1, 2, N), lambda h, i: (h, 0, 0)),
            pl.BlockSpec((1, BR, N), lambda h, i: (h, i, 0)),
            pl.BlockSpec((1, BR, N), lambda h, i: (h, i, 0)),
        ],
        out_specs=pl.BlockSpec((1, BR, N), lambda h, i: (h, i, 0)),
        out_shape=jax.ShapeDtypeStruct((H, N, N), jnp.float32),
    )(f, local_patten, long_range_patten)


# speedup vs baseline: 4.5275x; 1.0026x over previous
import jax
import jax.numpy as jnp
from jax.experimental import pallas as pl
from jax.experimental.pallas import tpu as pltpu

H, N, DH = 8, 2048, 128
BR = 256
ALPHA_MIX = 0.5


def _scores_kernel(x_ref, v_ref, f_ref):
    x = x_ref[0]
    v = v_ref[0, :, :, 0]
    f_ref[0] = jax.lax.dot_general(
        v, x, (((1,), (1,)), ((), ())),
        preferred_element_type=jnp.float32)


def _att_kernel(f_ref, loc_ref, lon_ref, out_ref):
    i = pl.program_id(1)
    f1 = f_ref[0, 0, pl.ds(i * BR, BR)]
    f2 = f_ref[0, 1, :]
    s = jax.nn.sigmoid(f1[:, None] + f2[None, :])
    e = jnp.exp(s)
    el = e * loc_ref[0]
    eg = e * lon_ref[0]
    dl = jnp.sum(el, axis=1, keepdims=True)
    dg = jnp.sum(eg, axis=1, keepdims=True)
    rl = (1.0 - ALPHA_MIX) / dl
    rg = ALPHA_MIX / dg
    out_ref[0] = el * rl + eg * rg


def kernel(local_patten, long_range_patten, weighted_X, V):
    f = pl.pallas_call(
        _scores_kernel,
        grid=(H,),
        in_specs=[
            pl.BlockSpec((1, N, DH), lambda h: (h, 0, 0)),
            pl.BlockSpec((1, 2, DH, 1), lambda h: (h, 0, 0, 0)),
        ],
        out_specs=pl.BlockSpec((1, 2, N), lambda h: (h, 0, 0)),
        out_shape=jax.ShapeDtypeStruct((H, 2, N), jnp.float32),
    )(weighted_X, V)

    return pl.pallas_call(
        _att_kernel,
        grid=(H, N // BR),
        in_specs=[
            pl.BlockSpec((1, 2, N), lambda h, i: (h, 0, 0)),
            pl.BlockSpec((1, BR, N), lambda h, i: (h, i, 0)),
            pl.BlockSpec((1, BR, N), lambda h, i: (h, i, 0)),
        ],
        out_specs=pl.BlockSpec((1, BR, N), lambda h, i: (h, i, 0)),
        out_shape=jax.ShapeDtypeStruct((H, N, N), jnp.float32),
        compiler_params=pltpu.CompilerParams(
            dimension_semantics=("parallel", "arbitrary")),
    )(f, local_patten, long_range_patten)


# speedup vs baseline: 4.9241x; 1.0876x over previous
import jax
import jax.numpy as jnp
from jax.experimental import pallas as pl
from jax.experimental.pallas import tpu as pltpu

H, N, DH = 8, 2048, 128
BR = 512
ALPHA_MIX = 0.5


def _scores_kernel(x_ref, v_ref, f_ref):
    x = x_ref[0]
    v = v_ref[0, :, :, 0]
    f_ref[0] = jax.lax.dot_general(
        v, x, (((1,), (1,)), ((), ())),
        preferred_element_type=jnp.float32)


def _att_kernel(f_ref, loc_ref, lon_ref, out_ref):
    i = pl.program_id(1)
    f1 = f_ref[0, 0, pl.ds(i * BR, BR)]
    f2 = f_ref[0, 1, :]
    s = jax.nn.sigmoid(f1[:, None] + f2[None, :])
    e = jnp.exp(s)
    el = e * loc_ref[0]
    eg = e * lon_ref[0]
    dl = jnp.sum(el, axis=1, keepdims=True)
    dg = jnp.sum(eg, axis=1, keepdims=True)
    rl = (1.0 - ALPHA_MIX) / dl
    rg = ALPHA_MIX / dg
    out_ref[0] = el * rl + eg * rg


def kernel(local_patten, long_range_patten, weighted_X, V):
    f = pl.pallas_call(
        _scores_kernel,
        grid=(H,),
        in_specs=[
            pl.BlockSpec((1, N, DH), lambda h: (h, 0, 0)),
            pl.BlockSpec((1, 2, DH, 1), lambda h: (h, 0, 0, 0)),
        ],
        out_specs=pl.BlockSpec((1, 2, N), lambda h: (h, 0, 0)),
        out_shape=jax.ShapeDtypeStruct((H, 2, N), jnp.float32),
    )(weighted_X, V)

    return pl.pallas_call(
        _att_kernel,
        grid=(H, N // BR),
        in_specs=[
            pl.BlockSpec((1, 2, N), lambda h, i: (h, 0, 0)),
            pl.BlockSpec((1, BR, N), lambda h, i: (h, i, 0)),
            pl.BlockSpec((1, BR, N), lambda h, i: (h, i, 0)),
        ],
        out_specs=pl.BlockSpec((1, BR, N), lambda h, i: (h, i, 0)),
        out_shape=jax.ShapeDtypeStruct((H, N, N), jnp.float32),
        compiler_params=pltpu.CompilerParams(
            dimension_semantics=("parallel", "arbitrary")),
    )(f, local_patten, long_range_patten)
